# baseline (speedup 1.0000x reference)
import functools

import jax
import jax.numpy as jnp
from jax.experimental import pallas as pl
from jax.experimental.pallas import tpu as pltpu


def _first_layer_body(h_ref, w_ref, b_ref, adj_ref, out_ref, adjq_ref,
                      support_ref):
    @pl.when(pl.program_id(0) == 0)
    def _():
        support_ref[...] = jnp.dot(h_ref[...], w_ref[...],
                                   preferred_element_type=jnp.float32)

    adjq_ref[...] = adj_ref[...].astype(jnp.float8_e4m3fn)
    acc = jnp.dot(adj_ref[...], support_ref[...],
                  preferred_element_type=jnp.float32)
    out_ref[...] = jnp.maximum(acc + b_ref[...], 0.0)


def _layer_body(h_ref, w_ref, b_ref, adj_ref, out_ref, support_ref, *, last):
    @pl.when(pl.program_id(0) == 0)
    def _():
        s = jnp.dot(h_ref[...], w_ref[...], preferred_element_type=jnp.float32)
        support_ref[...] = s.astype(jnp.bfloat16)

    acc = jnp.dot(adj_ref[...], support_ref[...],
                  preferred_element_type=jnp.float32)
    logits = acc + b_ref[...]
    if last:
        m = jnp.max(logits, axis=1, keepdims=True)
        lse = jnp.log(jnp.sum(jnp.exp(logits - m), axis=1, keepdims=True)) + m
        out_ref[...] = logits - lse
    else:
        out_ref[...] = jnp.maximum(logits, 0.0)


def _first_layer(x, adj, W, b, *, block):
    n, nin = x.shape
    nout = W.shape[1]
    grid = n // block
    return pl.pallas_call(
        _first_layer_body,
        grid=(grid,),
        in_specs=[
            pl.BlockSpec((n, nin), lambda i: (0, 0)),
            pl.BlockSpec((nin, nout), lambda i: (0, 0)),
            pl.BlockSpec((1, nout), lambda i: (0, 0)),
            pl.BlockSpec((block, n), lambda i: (i, 0)),
        ],
        out_specs=[
            pl.BlockSpec((block, nout), lambda i: (i, 0)),
            pl.BlockSpec((block, n), lambda i: (i, 0)),
        ],
        out_shape=[
            jax.ShapeDtypeStruct((n, nout), jnp.float32),
            jax.ShapeDtypeStruct((n, n), jnp.float8_e4m3fn),
        ],
        scratch_shapes=[pltpu.VMEM((n, nout), jnp.float32)],
        compiler_params=pltpu.CompilerParams(
            dimension_semantics=("arbitrary",),
        ),
    )(x, W, b.reshape(1, nout), adj)


def _layer(h, adj_q, W, b, *, last, block):
    n, nin = h.shape
    nout = W.shape[1]
    grid = n // block
    body = functools.partial(_layer_body, last=last)
    return pl.pallas_call(
        body,
        grid=(grid,),
        in_specs=[
            pl.BlockSpec((n, nin), lambda i: (0, 0)),
            pl.BlockSpec((nin, nout), lambda i: (0, 0)),
            pl.BlockSpec((1, nout), lambda i: (0, 0)),
            pl.BlockSpec((block, n), lambda i: (i, 0)),
        ],
        out_specs=pl.BlockSpec((block, nout), lambda i: (i, 0)),
        out_shape=jax.ShapeDtypeStruct((n, nout), jnp.float32),
        scratch_shapes=[pltpu.VMEM((n, nout), jnp.bfloat16)],
        compiler_params=pltpu.CompilerParams(
            dimension_semantics=("arbitrary",),
        ),
    )(h, W, b.reshape(1, nout), adj_q)


def kernel(x, adj, W1, b1, W2, b2, W3, b3, W4, b4, W5, b5, W6, b6):
    n = adj.shape[0]
    block1 = 200 if n % 200 == 0 else n
    block = 400 if n % 400 == 0 else n
    h, adj_q = _first_layer(x, adj, W1, b1, block=block1)
    for W, b in ((W2, b2), (W3, b3), (W4, b4), (W5, b5)):
        h = _layer(h, adj_q, W, b, last=False, block=block)
    return _layer(h, adj_q, W6, b6, last=True, block=block)


# speedup vs baseline: 1.0371x; 1.0371x over previous
import functools

import jax
import jax.numpy as jnp
from jax.experimental import pallas as pl
from jax.experimental.pallas import tpu as pltpu


def _first_layer_body(h_ref, w_ref, b_ref, adj_ref, out_ref, adjq_ref,
                      support_ref):
    @pl.when(pl.program_id(0) == 0)
    def _():
        support_ref[...] = jnp.dot(h_ref[...], w_ref[...],
                                   preferred_element_type=jnp.float32)

    adjq_ref[...] = adj_ref[...].astype(jnp.float8_e4m3fn)
    acc = jnp.dot(adj_ref[...], support_ref[...],
                  preferred_element_type=jnp.float32)
    out_ref[...] = jnp.maximum(acc + b_ref[...], 0.0)


def _layer_body(h_ref, w_ref, b_ref, adj_ref, out_ref, support_ref, *, last):
    @pl.when(pl.program_id(0) == 0)
    def _():
        s = jnp.dot(h_ref[...], w_ref[...], preferred_element_type=jnp.float32)
        support_ref[...] = s.astype(jnp.bfloat16)

    acc = jnp.dot(adj_ref[...], support_ref[...],
                  preferred_element_type=jnp.float32)
    logits = acc + b_ref[...]
    if last:
        m = jnp.max(logits, axis=1, keepdims=True)
        lse = jnp.log(jnp.sum(jnp.exp(logits - m), axis=1, keepdims=True)) + m
        out_ref[...] = logits - lse
    else:
        out_ref[...] = jnp.maximum(logits, 0.0)


def _first_layer(x, adj, W, b, *, block):
    n, nin = x.shape
    nout = W.shape[1]
    grid = n // block
    return pl.pallas_call(
        _first_layer_body,
        grid=(grid,),
        in_specs=[
            pl.BlockSpec((n, nin), lambda i: (0, 0)),
            pl.BlockSpec((nin, nout), lambda i: (0, 0)),
            pl.BlockSpec((1, nout), lambda i: (0, 0)),
            pl.BlockSpec((block, n), lambda i: (i, 0)),
        ],
        out_specs=[
            pl.BlockSpec((block, nout), lambda i: (i, 0)),
            pl.BlockSpec((block, n), lambda i: (i, 0)),
        ],
        out_shape=[
            jax.ShapeDtypeStruct((n, nout), jnp.float32),
            jax.ShapeDtypeStruct((n, n), jnp.float8_e4m3fn),
        ],
        scratch_shapes=[pltpu.VMEM((n, nout), jnp.float32)],
        compiler_params=pltpu.CompilerParams(
            dimension_semantics=("arbitrary",),
        ),
    )(x, W, b.reshape(1, nout), adj)


def _layer(h, adj_q, W, b, *, last, block):
    n, nin = h.shape
    nout = W.shape[1]
    grid = n // block
    body = functools.partial(_layer_body, last=last)
    return pl.pallas_call(
        body,
        grid=(grid,),
        in_specs=[
            pl.BlockSpec((n, nin), lambda i: (0, 0)),
            pl.BlockSpec((nin, nout), lambda i: (0, 0)),
            pl.BlockSpec((1, nout), lambda i: (0, 0)),
            pl.BlockSpec((block, n), lambda i: (i, 0)),
        ],
        out_specs=pl.BlockSpec((block, nout), lambda i: (i, 0)),
        out_shape=jax.ShapeDtypeStruct((n, nout), jnp.float32),
        scratch_shapes=[pltpu.VMEM((n, nout), jnp.bfloat16)],
        compiler_params=pltpu.CompilerParams(
            dimension_semantics=("arbitrary",),
        ),
    )(h, W, b.reshape(1, nout), adj_q)


def kernel(x, adj, W1, b1, W2, b2, W3, b3, W4, b4, W5, b5, W6, b6):
    n = adj.shape[0]
    block1 = 400 if n % 400 == 0 else n
    block = 1000 if n % 1000 == 0 else n
    h, adj_q = _first_layer(x, adj, W1, b1, block=block1)
    for W, b in ((W2, b2), (W3, b3), (W4, b4), (W5, b5)):
        h = _layer(h, adj_q, W, b, last=False, block=block)
    return _layer(h, adj_q, W6, b6, last=True, block=block)


# speedup vs baseline: 1.2055x; 1.1624x over previous
import functools

import jax
import jax.numpy as jnp
from jax.experimental import pallas as pl
from jax.experimental.pallas import tpu as pltpu

_FP8 = jnp.float8_e4m3fn


def _first_layer_body(h_ref, w_ref, b_ref, adj_ref, out_ref, adjq_ref,
                      rowsum_ref, support_ref):
    @pl.when(pl.program_id(0) == 0)
    def _():
        support_ref[...] = jnp.dot(h_ref[...], w_ref[...],
                                   preferred_element_type=jnp.float32)

    aq = adj_ref[...].astype(_FP8)
    adjq_ref[...] = aq
    rowsum_ref[...] = jnp.sum(aq.astype(jnp.float32), axis=1, keepdims=True)
    acc = jnp.dot(adj_ref[...], support_ref[...],
                  preferred_element_type=jnp.float32)
    out_ref[...] = jnp.maximum(acc + b_ref[...], 0.0)


def _layer_body(h_ref, w_ref, b_ref, adj_ref, rowsum_ref, out_ref,
                support_ref, sc_ref, dmu_ref, *, last):
    @pl.when(pl.program_id(0) == 0)
    def _():
        s = jnp.dot(h_ref[...], w_ref[...], preferred_element_type=jnp.float32)
        sc = jnp.max(jnp.abs(s), axis=0, keepdims=True) * (1.0 / 240.0)
        sc = jnp.maximum(sc, 1e-30)
        sc_ref[...] = sc
        s_scaled = s * (1.0 / sc)
        sq = s_scaled.astype(_FP8)
        support_ref[...] = sq
        dmu_ref[...] = jnp.mean(s_scaled - sq.astype(jnp.float32), axis=0,
                                keepdims=True)

    acc = jnp.dot(adj_ref[...], support_ref[...],
                  preferred_element_type=jnp.float32)
    acc = acc + rowsum_ref[...] * dmu_ref[...]
    logits = acc * sc_ref[...] + b_ref[...]
    if last:
        m = jnp.max(logits, axis=1, keepdims=True)
        lse = jnp.log(jnp.sum(jnp.exp(logits - m), axis=1, keepdims=True)) + m
        out_ref[...] = logits - lse
    else:
        out_ref[...] = jnp.maximum(logits, 0.0)


def _first_layer(x, adj, W, b, *, block):
    n, nin = x.shape
    nout = W.shape[1]
    grid = n // block
    return pl.pallas_call(
        _first_layer_body,
        grid=(grid,),
        in_specs=[
            pl.BlockSpec((n, nin), lambda i: (0, 0)),
            pl.BlockSpec((nin, nout), lambda i: (0, 0)),
            pl.BlockSpec((1, nout), lambda i: (0, 0)),
            pl.BlockSpec((block, n), lambda i: (i, 0)),
        ],
        out_specs=[
            pl.BlockSpec((block, nout), lambda i: (i, 0)),
            pl.BlockSpec((block, n), lambda i: (i, 0)),
            pl.BlockSpec((block, 1), lambda i: (i, 0)),
        ],
        out_shape=[
            jax.ShapeDtypeStruct((n, nout), jnp.float32),
            jax.ShapeDtypeStruct((n, n), _FP8),
            jax.ShapeDtypeStruct((n, 1), jnp.float32),
        ],
        scratch_shapes=[pltpu.VMEM((n, nout), jnp.float32)],
        compiler_params=pltpu.CompilerParams(
            dimension_semantics=("arbitrary",),
        ),
    )(x, W, b.reshape(1, nout), adj)


def _layer(h, adj_q, rowsum, W, b, *, last, block):
    n, nin = h.shape
    nout = W.shape[1]
    grid = n // block
    body = functools.partial(_layer_body, last=last)
    return pl.pallas_call(
        body,
        grid=(grid,),
        in_specs=[
            pl.BlockSpec((n, nin), lambda i: (0, 0)),
            pl.BlockSpec((nin, nout), lambda i: (0, 0)),
            pl.BlockSpec((1, nout), lambda i: (0, 0)),
            pl.BlockSpec((block, n), lambda i: (i, 0)),
            pl.BlockSpec((block, 1), lambda i: (i, 0)),
        ],
        out_specs=pl.BlockSpec((block, nout), lambda i: (i, 0)),
        out_shape=jax.ShapeDtypeStruct((n, nout), jnp.float32),
        scratch_shapes=[
            pltpu.VMEM((n, nout), _FP8),
            pltpu.VMEM((1, nout), jnp.float32),
            pltpu.VMEM((1, nout), jnp.float32),
        ],
        compiler_params=pltpu.CompilerParams(
            dimension_semantics=("arbitrary",),
        ),
    )(h, W, b.reshape(1, nout), adj_q, rowsum)


def kernel(x, adj, W1, b1, W2, b2, W3, b3, W4, b4, W5, b5, W6, b6):
    n = adj.shape[0]
    block1 = 400 if n % 400 == 0 else n
    block = 1000 if n % 1000 == 0 else n
    h, adj_q, rowsum = _first_layer(x, adj, W1, b1, block=block1)
    for W, b in ((W2, b2), (W3, b3), (W4, b4), (W5, b5)):
        h = _layer(h, adj_q, rowsum, W, b, last=False, block=block)
    return _layer(h, adj_q, rowsum, W6, b6, last=True, block=block)
